# Initial kernel scaffold; baseline (speedup 1.0000x reference)
import jax
import jax.numpy as jnp
from jax.experimental import pallas as pl


def kernel(x, edge_index, W1, b1, W2, b2, M1w, M1b, M2w, M2b, M3w, M3b):
    raise NotImplementedError("write your pallas kernel here")


# speedup vs baseline: 14.3801x; 14.3801x over previous
import functools

import jax
import jax.numpy as jnp
from jax import lax
from jax.experimental import pallas as pl
from jax.experimental.pallas import tpu as pltpu
from jax.experimental.pallas import tpu_sc as plsc

_NC, _NS = 2, 16
_NW = _NC * _NS
_K = 80
_SQ_ITERS = 17
_P = 128
_CW = 16


def _leaky(v):
    return jnp.where(v >= 0, v, 0.2 * v)


def _sn_body(w_ref, out_ref):
    W = w_ref[0]
    r = lax.broadcasted_iota(jnp.int32, (_P, _P), 0)
    c = lax.broadcasted_iota(jnp.int32, (_P, _P), 1)
    eye = r == c
    B = lax.dot_general(W, W, (((1,), (1,)), ((), ())),
                        preferred_element_type=jnp.float32)
    tr0 = jnp.sum(jnp.where(eye, B, 0.0))
    Bh = B / (tr0 + 1e-30)

    def sq(_, M):
        M2 = jnp.dot(M, M, preferred_element_type=jnp.float32)
        t = jnp.sum(jnp.where(eye, M2, 0.0))
        return M2 / (t + 1e-30)

    C = lax.fori_loop(0, _SQ_ITERS, sq, Bh)
    lam = jnp.sum(C * B)
    sigma = jnp.sqrt(jnp.maximum(lam, 0.0))
    out_ref[0] = W / (sigma + 1e-12)


def _sn_all(Ws):
    nw = Ws.shape[0]
    return pl.pallas_call(
        _sn_body,
        grid=(nw,),
        in_specs=[pl.BlockSpec((1, _P, _P), lambda i: (i, 0, 0))],
        out_specs=pl.BlockSpec((1, _P, _P), lambda i: (i, 0, 0)),
        out_shape=jax.ShapeDtypeStruct((nw, _P, _P), jnp.float32),
    )(Ws)


def _npad(N):
    step = _NS * 8
    return ((N + step - 1) // step) * step


def _deg_counts(dst, N, E):
    e_per_w = E // _NW
    NR = _npad(N) // 128 * 1
    if NR * 128 < N:
        NR = (N + 127) // 128
    NR = ((NR + 7) // 8) * 8
    zeros = jnp.zeros((NR, 128), jnp.float32)
    mesh = plsc.VectorSubcoreMesh(core_axis_name="c", subcore_axis_name="s",
                                  num_cores=_NC, num_subcores=_NS)
    rows_per_out_tile = 8
    n_out_tiles = NR // rows_per_out_tile

    @functools.partial(
        pl.kernel,
        out_type=jax.ShapeDtypeStruct((_NC, NR, 128), jnp.float32),
        mesh=mesh,
        scratch_types=[
            pltpu.VMEM((e_per_w,), jnp.int32),
            pltpu.VMEM((NR, 128), jnp.float32),
            pltpu.VMEM((NR,), jnp.int32),
            pltpu.VMEM_SHARED((NR, 128), jnp.float32),
        ],
        compiler_params=pltpu.CompilerParams(needs_layout_passes=False),
    )
    def deg_kernel(dst_hbm, z_hbm, out_hbm, dst_v, cnt_v, iden_v, acc_sh):
        c = lax.axis_index("c")
        s = lax.axis_index("s")
        pltpu.sync_copy(z_hbm, cnt_v)

        @pl.when(s == 0)
        def _():
            pltpu.sync_copy(z_hbm, acc_sh)

        base = pl.multiple_of((c * _NS + s) * e_per_w, 8)
        pltpu.sync_copy(dst_hbm.at[pl.ds(base, e_per_w)], dst_v)

        def fill_iota(i, carry):
            iden_v[pl.ds(i * 16, 16)] = lax.iota(jnp.int32, 16) + i * 16
            return carry

        lax.fori_loop(0, NR // 16, fill_iota, 0)

        def body(j, carry):
            idx = dst_v[pl.ds(j * 16, 16)]
            cnts, last = plsc.scan_count(idx)
            row = lax.shift_right_logical(idx, 7)
            col = lax.bitwise_and(idx, 127)
            plsc.addupdate_scatter(cnt_v, [row, col],
                                   cnts.astype(jnp.float32), mask=last)
            return carry

        lax.fori_loop(0, e_per_w // 16, body, 0)
        plsc.subcore_barrier()
        pltpu.sync_copy(cnt_v, acc_sh.at[iden_v], add=True)
        plsc.subcore_barrier()

        @pl.when(s < n_out_tiles)
        def _():
            pltpu.sync_copy(
                acc_sh.at[pl.ds(s * rows_per_out_tile, rows_per_out_tile)],
                out_hbm.at[c, pl.ds(s * rows_per_out_tile,
                                    rows_per_out_tile)])

    return deg_kernel(dst, zeros)


def _aggregate(hp, src, dst, N, E, D):
    e_per_w = E // _NW
    n_chunks = e_per_w // _K
    NP = _npad(N)
    rpt = NP // _NS
    zeros = jnp.zeros((rpt, D), jnp.float32)
    mesh = plsc.VectorSubcoreMesh(core_axis_name="c", subcore_axis_name="s",
                                  num_cores=_NC, num_subcores=_NS)

    @functools.partial(
        pl.kernel,
        out_type=jax.ShapeDtypeStruct((_NC, NP, D), jnp.float32),
        mesh=mesh,
        scratch_types=[
            pltpu.VMEM((_K,), jnp.int32),
            pltpu.VMEM((_K,), jnp.int32),
            pltpu.VMEM((_K, D), jnp.float32),
            pltpu.VMEM_SHARED((NP, D), jnp.float32),
            pltpu.SemaphoreType.DMA,
        ],
    )
    def agg_kernel(hp_hbm, src_hbm, dst_hbm, z_hbm, out_hbm,
                   src_v, dst_v, rows_v, acc_sh, sem):
        c = lax.axis_index("c")
        s = lax.axis_index("s")
        pltpu.sync_copy(z_hbm, acc_sh.at[pl.ds(s * rpt, rpt)])
        plsc.subcore_barrier()
        ebase = (c * _NS + s) * e_per_w

        def body(j, carry):
            base = pl.multiple_of(ebase + j * _K, 8)
            pltpu.sync_copy(src_hbm.at[pl.ds(base, _K)], src_v)
            pltpu.sync_copy(dst_hbm.at[pl.ds(base, _K)], dst_v)
            pltpu.async_copy(hp_hbm.at[src_v], rows_v, sem).wait()
            pltpu.sync_copy(rows_v, acc_sh.at[dst_v], add=True)
            return carry

        lax.fori_loop(0, n_chunks, body, 0)
        plsc.subcore_barrier()
        pltpu.sync_copy(acc_sh.at[pl.ds(s * rpt, rpt)],
                        out_hbm.at[c, pl.ds(s * rpt, rpt)])

    return agg_kernel(hp, src, dst, zeros)


def _dis_from_cnt(cnt):
    deg = cnt[0, :, 0] + cnt[1, :, 0] + 1.0
    return lax.rsqrt(deg)[:, None]


def _mm_scale_body(x_ref, w_ref, cnt_ref, out_ref):
    dis = _dis_from_cnt(cnt_ref[...])
    h = lax.dot_general(x_ref[...], w_ref[...], (((1,), (1,)), ((), ())),
                        preferred_element_type=jnp.float32)
    out_ref[...] = h * dis


def _layer1_premul(x, W1n, cnt, blk):
    N, D = x.shape
    grid = N // blk
    return pl.pallas_call(
        _mm_scale_body,
        grid=(grid,),
        in_specs=[
            pl.BlockSpec((blk, D), lambda i: (i, 0)),
            pl.BlockSpec(W1n.shape, lambda i: (0, 0)),
            pl.BlockSpec((_NC, blk, 1), lambda i: (0, i, 0)),
        ],
        out_specs=pl.BlockSpec((blk, W1n.shape[0]), lambda i: (i, 0)),
        out_shape=jax.ShapeDtypeStruct((N, W1n.shape[0]), jnp.float32),
    )(x, W1n, cnt)


def _layer2_body(p_ref, hp_ref, cnt_ref, b_ref, w_ref, out_ref):
    dis = _dis_from_cnt(cnt_ref[...])
    agg = p_ref[0] + p_ref[1] + hp_ref[...]
    o1 = _leaky(agg * dis + b_ref[...])
    h2 = lax.dot_general(o1, w_ref[...], (((1,), (1,)), ((), ())),
                         preferred_element_type=jnp.float32)
    out_ref[...] = h2 * dis


def _layer2_premul(p, hp, cnt, b1, W2n, blk):
    N, D = hp.shape
    grid = N // blk
    return pl.pallas_call(
        _layer2_body,
        grid=(grid,),
        in_specs=[
            pl.BlockSpec((_NC, blk, D), lambda i: (0, i, 0)),
            pl.BlockSpec((blk, D), lambda i: (i, 0)),
            pl.BlockSpec((_NC, blk, 1), lambda i: (0, i, 0)),
            pl.BlockSpec((1, D), lambda i: (0, 0)),
            pl.BlockSpec(W2n.shape, lambda i: (0, 0)),
        ],
        out_specs=pl.BlockSpec((blk, W2n.shape[0]), lambda i: (i, 0)),
        out_shape=jax.ShapeDtypeStruct((N, W2n.shape[0]), jnp.float32),
    )(p, hp, cnt, b1, W2n)


def _final_body(p_ref, hp_ref, cnt_ref, b_ref, m1_ref, m2_ref, m3_ref,
                mb_ref, out_ref, acc_ref, *, n_nodes):
    i = pl.program_id(0)

    @pl.when(i == 0)
    def _():
        acc_ref[...] = jnp.zeros_like(acc_ref)

    dis = _dis_from_cnt(cnt_ref[...])
    o2 = _leaky((p_ref[0] + p_ref[1] + hp_ref[...]) * dis + b_ref[...])
    acc_ref[...] += jnp.sum(o2, axis=0, keepdims=True)

    @pl.when(i == pl.num_programs(0) - 1)
    def _():
        g = acc_ref[...] / n_nodes
        a = _leaky(lax.dot_general(g, m1_ref[...], (((1,), (1,)), ((), ())),
                                   preferred_element_type=jnp.float32)
                   + mb_ref[0:1])
        a = _leaky(lax.dot_general(a, m2_ref[...], (((1,), (1,)), ((), ())),
                                   preferred_element_type=jnp.float32)
                   + mb_ref[1:2])
        a = (lax.dot_general(a, m3_ref[...], (((1,), (1,)), ((), ())),
                             preferred_element_type=jnp.float32)
             + mb_ref[2:3])
        out_ref[...] = a[0:1, 0:1]


def _final(p, hp, cnt, b2, M1n, M2n, M3n, mb, blk):
    N, D = hp.shape
    grid = N // blk
    return pl.pallas_call(
        functools.partial(_final_body, n_nodes=N),
        grid=(grid,),
        in_specs=[
            pl.BlockSpec((_NC, blk, D), lambda i: (0, i, 0)),
            pl.BlockSpec((blk, D), lambda i: (i, 0)),
            pl.BlockSpec((_NC, blk, 1), lambda i: (0, i, 0)),
            pl.BlockSpec((1, D), lambda i: (0, 0)),
            pl.BlockSpec((_P, _P), lambda i: (0, 0)),
            pl.BlockSpec((_P, _P), lambda i: (0, 0)),
            pl.BlockSpec((_P, _P), lambda i: (0, 0)),
            pl.BlockSpec((3, _P), lambda i: (0, 0)),
        ],
        out_specs=pl.BlockSpec((1, 1), lambda i: (0, 0)),
        out_shape=jax.ShapeDtypeStruct((1, 1), jnp.float32),
        scratch_shapes=[pltpu.VMEM((1, _P), jnp.float32)],
    )(p, hp, cnt, b2, M1n, M2n, M3n, mb)


def kernel(x, edge_index, W1, b1, W2, b2, M1w, M1b, M2w, M2b, M3w, M3b):
    N, D = x.shape
    E = edge_index.shape[1]
    f32 = jnp.float32
    x = x.astype(f32)

    def padw(w):
        return jnp.zeros((_P, _P), f32).at[:w.shape[0], :w.shape[1]].set(w)

    Ws = jnp.stack([padw(W1), padw(W2), padw(M1w), padw(M2w), padw(M3w)])
    Wn = _sn_all(Ws)
    W1n = Wn[0, :W1.shape[0], :W1.shape[1]]
    W2n = Wn[1, :W2.shape[0], :W2.shape[1]]
    M1n, M2n, M3n = Wn[2], Wn[3], Wn[4]
    mb = jnp.zeros((3, _P), f32)
    mb = mb.at[0, :M1b.shape[0]].set(M1b)
    mb = mb.at[1, :M2b.shape[0]].set(M2b)
    mb = mb.at[2, :M3b.shape[0]].set(M3b)

    blk = 2000
    src = edge_index[0]
    dst = edge_index[1]
    cnt = _deg_counts(dst, N, E).reshape(_NC, -1, 1)
    h1p = _layer1_premul(x, W1n, cnt, blk)
    p1 = _aggregate(h1p, src, dst, N, E, W1n.shape[0])
    h2p = _layer2_premul(p1, h1p, cnt, b1[None, :], W2n, blk)
    p2 = _aggregate(h2p, src, dst, N, E, W2n.shape[0])
    return _final(p2, h2p, cnt, b2[None, :], M1n, M2n, M3n, mb, blk)


# speedup vs baseline: 26.2972x; 1.8287x over previous
import functools

import jax
import jax.numpy as jnp
from jax import lax
from jax.experimental import pallas as pl
from jax.experimental.pallas import tpu as pltpu
from jax.experimental.pallas import tpu_sc as plsc

_NC, _NS = 2, 16
_NW = _NC * _NS
_K = 80
_SQ_ITERS = 17
_P = 128
_CW = 16


def _leaky(v):
    return jnp.where(v >= 0, v, 0.2 * v)


def _sn_body(w_ref, out_ref):
    W = w_ref[0]
    r = lax.broadcasted_iota(jnp.int32, (_P, _P), 0)
    c = lax.broadcasted_iota(jnp.int32, (_P, _P), 1)
    eye = r == c
    B = lax.dot_general(W, W, (((1,), (1,)), ((), ())),
                        preferred_element_type=jnp.float32)
    tr0 = jnp.sum(jnp.where(eye, B, 0.0))
    Bh = B / (tr0 + 1e-30)

    def sq(_, M):
        M2 = jnp.dot(M, M, preferred_element_type=jnp.float32)
        t = jnp.sum(jnp.where(eye, M2, 0.0))
        return M2 / (t + 1e-30)

    C = lax.fori_loop(0, _SQ_ITERS, sq, Bh)
    lam = jnp.sum(C * B)
    sigma = jnp.sqrt(jnp.maximum(lam, 0.0))
    out_ref[0] = W / (sigma + 1e-12)


def _sn_all(Ws):
    nw = Ws.shape[0]
    return pl.pallas_call(
        _sn_body,
        grid=(nw,),
        in_specs=[pl.BlockSpec((1, _P, _P), lambda i: (i, 0, 0))],
        out_specs=pl.BlockSpec((1, _P, _P), lambda i: (i, 0, 0)),
        out_shape=jax.ShapeDtypeStruct((nw, _P, _P), jnp.float32),
    )(Ws)


def _npad(N):
    step = _NS * 8
    return ((N + step - 1) // step) * step


def _deg_counts(dst, N, E):
    e_per_w = E // _NW
    NR = _npad(N) // 128 * 1
    if NR * 128 < N:
        NR = (N + 127) // 128
    NR = ((NR + 7) // 8) * 8
    zeros = jnp.zeros((NR, 128), jnp.float32)
    mesh = plsc.VectorSubcoreMesh(core_axis_name="c", subcore_axis_name="s",
                                  num_cores=_NC, num_subcores=_NS)
    rows_per_out_tile = 8
    n_out_tiles = NR // rows_per_out_tile

    @functools.partial(
        pl.kernel,
        out_type=jax.ShapeDtypeStruct((_NC, NR, 128), jnp.float32),
        mesh=mesh,
        scratch_types=[
            pltpu.VMEM((e_per_w,), jnp.int32),
            pltpu.VMEM((NR, 128), jnp.float32),
            pltpu.VMEM((NR,), jnp.int32),
            pltpu.VMEM_SHARED((NR, 128), jnp.float32),
        ],
        compiler_params=pltpu.CompilerParams(needs_layout_passes=False),
    )
    def deg_kernel(dst_hbm, z_hbm, out_hbm, dst_v, cnt_v, iden_v, acc_sh):
        c = lax.axis_index("c")
        s = lax.axis_index("s")
        pltpu.sync_copy(z_hbm, cnt_v)

        @pl.when(s == 0)
        def _():
            pltpu.sync_copy(z_hbm, acc_sh)

        base = pl.multiple_of((c * _NS + s) * e_per_w, 8)
        pltpu.sync_copy(dst_hbm.at[pl.ds(base, e_per_w)], dst_v)

        def fill_iota(i, carry):
            iden_v[pl.ds(i * 16, 16)] = lax.iota(jnp.int32, 16) + i * 16
            return carry

        lax.fori_loop(0, NR // 16, fill_iota, 0)

        def body(j, carry):
            idx = dst_v[pl.ds(j * 16, 16)]
            cnts, last = plsc.scan_count(idx)
            row = lax.shift_right_logical(idx, 7)
            col = lax.bitwise_and(idx, 127)
            plsc.addupdate_scatter(cnt_v, [row, col],
                                   cnts.astype(jnp.float32), mask=last)
            return carry

        lax.fori_loop(0, e_per_w // 16, body, 0)
        plsc.subcore_barrier()
        pltpu.sync_copy(cnt_v, acc_sh.at[iden_v], add=True)
        plsc.subcore_barrier()

        @pl.when(s < n_out_tiles)
        def _():
            pltpu.sync_copy(
                acc_sh.at[pl.ds(s * rows_per_out_tile, rows_per_out_tile)],
                out_hbm.at[c, pl.ds(s * rows_per_out_tile,
                                    rows_per_out_tile)])

    return deg_kernel(dst, zeros)


def _aggregate(hp, src, dst, N, E, D):
    e_per_w = E // _NW
    n_chunks = e_per_w // _K
    assert n_chunks >= 2
    n_pairs = (n_chunks + 1) // 2
    NP = _npad(N)
    rpt = NP // _NS
    zeros = jnp.zeros((rpt, D), jnp.float32)
    mesh = plsc.VectorSubcoreMesh(core_axis_name="c", subcore_axis_name="s",
                                  num_cores=_NC, num_subcores=_NS)

    @functools.partial(
        pl.kernel,
        out_type=jax.ShapeDtypeStruct((_NC, NP, D), jnp.float32),
        mesh=mesh,
        scratch_types=[
            pltpu.VMEM((_K,), jnp.int32),
            pltpu.VMEM((_K,), jnp.int32),
            pltpu.VMEM((_K,), jnp.int32),
            pltpu.VMEM((_K,), jnp.int32),
            pltpu.VMEM((_K, D), jnp.float32),
            pltpu.VMEM((_K, D), jnp.float32),
            pltpu.VMEM_SHARED((NP, D), jnp.float32),
            pltpu.SemaphoreType.DMA,
            pltpu.SemaphoreType.DMA,
            pltpu.SemaphoreType.DMA,
            pltpu.SemaphoreType.DMA,
            pltpu.SemaphoreType.DMA,
            pltpu.SemaphoreType.DMA,
        ],
    )
    def agg_kernel(hp_hbm, src_hbm, dst_hbm, z_hbm, out_hbm,
                   src_a, dst_a, src_b, dst_b, rows_a, rows_b, acc_sh,
                   sem_sa, sem_da, sem_sb, sem_db, sem_ga, sem_gb):
        c = lax.axis_index("c")
        s = lax.axis_index("s")
        ebase = (c * _NS + s) * e_per_w

        def i_start(j, sv, dv, ss, ds_):
            base = pl.multiple_of(ebase + j * _K, 8)
            pltpu.async_copy(src_hbm.at[pl.ds(base, _K)], sv, ss)
            pltpu.async_copy(dst_hbm.at[pl.ds(base, _K)], dv, ds_)

        def i_wait(sv, dv, ss, ds_):
            pltpu.make_async_copy(src_hbm.at[pl.ds(0, _K)], sv, ss).wait()
            pltpu.make_async_copy(dst_hbm.at[pl.ds(0, _K)], dv, ds_).wait()

        def g_start(sv, rows, sem):
            pltpu.async_copy(hp_hbm.at[sv], rows, sem)

        def g_wait(rows, sem):
            pltpu.make_async_copy(hp_hbm.at[src_a], rows, sem).wait()

        bufs = ((src_a, dst_a, sem_sa, sem_da, rows_a, sem_ga),
                (src_b, dst_b, sem_sb, sem_db, rows_b, sem_gb))

        i_start(0, src_a, dst_a, sem_sa, sem_da)
        i_start(1, src_b, dst_b, sem_sb, sem_db)
        pltpu.sync_copy(z_hbm, acc_sh.at[pl.ds(s * rpt, rpt)])
        i_wait(src_a, dst_a, sem_sa, sem_da)
        plsc.subcore_barrier()
        g_start(src_a, rows_a, sem_ga)

        def body(i, carry):
            j0 = i * 2
            for p in range(2):
                j = j0 + p
                cur = bufs[p]
                oth = bufs[1 - p]

                @pl.when(j < n_chunks)
                def _():
                    @pl.when(j + 1 < n_chunks)
                    def _():
                        i_wait(oth[0], oth[1], oth[2], oth[3])
                        g_start(oth[0], oth[4], oth[5])

                    g_wait(cur[4], cur[5])
                    pltpu.sync_copy(cur[4], acc_sh.at[cur[1]], add=True)

                    @pl.when(j + 2 < n_chunks)
                    def _():
                        i_start(j + 2, cur[0], cur[1], cur[2], cur[3])

            return carry

        lax.fori_loop(0, n_pairs, body, 0)
        plsc.subcore_barrier()
        pltpu.sync_copy(acc_sh.at[pl.ds(s * rpt, rpt)],
                        out_hbm.at[c, pl.ds(s * rpt, rpt)])

    return agg_kernel(hp, src, dst, zeros)


def _dis_from_cnt(cnt):
    deg = cnt[0, :, 0] + cnt[1, :, 0] + 1.0
    return lax.rsqrt(deg)[:, None]


def _mm_scale_body(x_ref, w_ref, cnt_ref, out_ref):
    dis = _dis_from_cnt(cnt_ref[...])
    h = lax.dot_general(x_ref[...], w_ref[...], (((1,), (1,)), ((), ())),
                        preferred_element_type=jnp.float32)
    out_ref[...] = h * dis


def _layer1_premul(x, W1n, cnt, blk):
    N, D = x.shape
    grid = N // blk
    return pl.pallas_call(
        _mm_scale_body,
        grid=(grid,),
        in_specs=[
            pl.BlockSpec((blk, D), lambda i: (i, 0)),
            pl.BlockSpec(W1n.shape, lambda i: (0, 0)),
            pl.BlockSpec((_NC, blk, 1), lambda i: (0, i, 0)),
        ],
        out_specs=pl.BlockSpec((blk, W1n.shape[0]), lambda i: (i, 0)),
        out_shape=jax.ShapeDtypeStruct((N, W1n.shape[0]), jnp.float32),
    )(x, W1n, cnt)


def _layer2_body(p_ref, hp_ref, cnt_ref, b_ref, w_ref, out_ref):
    dis = _dis_from_cnt(cnt_ref[...])
    agg = p_ref[0] + p_ref[1] + hp_ref[...]
    o1 = _leaky(agg * dis + b_ref[...])
    h2 = lax.dot_general(o1, w_ref[...], (((1,), (1,)), ((), ())),
                         preferred_element_type=jnp.float32)
    out_ref[...] = h2 * dis


def _layer2_premul(p, hp, cnt, b1, W2n, blk):
    N, D = hp.shape
    grid = N // blk
    return pl.pallas_call(
        _layer2_body,
        grid=(grid,),
        in_specs=[
            pl.BlockSpec((_NC, blk, D), lambda i: (0, i, 0)),
            pl.BlockSpec((blk, D), lambda i: (i, 0)),
            pl.BlockSpec((_NC, blk, 1), lambda i: (0, i, 0)),
            pl.BlockSpec((1, D), lambda i: (0, 0)),
            pl.BlockSpec(W2n.shape, lambda i: (0, 0)),
        ],
        out_specs=pl.BlockSpec((blk, W2n.shape[0]), lambda i: (i, 0)),
        out_shape=jax.ShapeDtypeStruct((N, W2n.shape[0]), jnp.float32),
    )(p, hp, cnt, b1, W2n)


def _final_body(p_ref, hp_ref, cnt_ref, b_ref, m1_ref, m2_ref, m3_ref,
                mb_ref, out_ref, acc_ref, *, n_nodes):
    i = pl.program_id(0)

    @pl.when(i == 0)
    def _():
        acc_ref[...] = jnp.zeros_like(acc_ref)

    dis = _dis_from_cnt(cnt_ref[...])
    o2 = _leaky((p_ref[0] + p_ref[1] + hp_ref[...]) * dis + b_ref[...])
    acc_ref[...] += jnp.sum(o2, axis=0, keepdims=True)

    @pl.when(i == pl.num_programs(0) - 1)
    def _():
        g = acc_ref[...] / n_nodes
        a = _leaky(lax.dot_general(g, m1_ref[...], (((1,), (1,)), ((), ())),
                                   preferred_element_type=jnp.float32)
                   + mb_ref[0:1])
        a = _leaky(lax.dot_general(a, m2_ref[...], (((1,), (1,)), ((), ())),
                                   preferred_element_type=jnp.float32)
                   + mb_ref[1:2])
        a = (lax.dot_general(a, m3_ref[...], (((1,), (1,)), ((), ())),
                             preferred_element_type=jnp.float32)
             + mb_ref[2:3])
        out_ref[...] = a[0:1, 0:1]


def _final(p, hp, cnt, b2, M1n, M2n, M3n, mb, blk):
    N, D = hp.shape
    grid = N // blk
    return pl.pallas_call(
        functools.partial(_final_body, n_nodes=N),
        grid=(grid,),
        in_specs=[
            pl.BlockSpec((_NC, blk, D), lambda i: (0, i, 0)),
            pl.BlockSpec((blk, D), lambda i: (i, 0)),
            pl.BlockSpec((_NC, blk, 1), lambda i: (0, i, 0)),
            pl.BlockSpec((1, D), lambda i: (0, 0)),
            pl.BlockSpec((_P, _P), lambda i: (0, 0)),
            pl.BlockSpec((_P, _P), lambda i: (0, 0)),
            pl.BlockSpec((_P, _P), lambda i: (0, 0)),
            pl.BlockSpec((3, _P), lambda i: (0, 0)),
        ],
        out_specs=pl.BlockSpec((1, 1), lambda i: (0, 0)),
        out_shape=jax.ShapeDtypeStruct((1, 1), jnp.float32),
        scratch_shapes=[pltpu.VMEM((1, _P), jnp.float32)],
    )(p, hp, cnt, b2, M1n, M2n, M3n, mb)


def kernel(x, edge_index, W1, b1, W2, b2, M1w, M1b, M2w, M2b, M3w, M3b):
    N, D = x.shape
    E = edge_index.shape[1]
    f32 = jnp.float32
    x = x.astype(f32)

    def padw(w):
        return jnp.zeros((_P, _P), f32).at[:w.shape[0], :w.shape[1]].set(w)

    Ws = jnp.stack([padw(W1), padw(W2), padw(M1w), padw(M2w), padw(M3w)])
    Wn = _sn_all(Ws)
    W1n = Wn[0, :W1.shape[0], :W1.shape[1]]
    W2n = Wn[1, :W2.shape[0], :W2.shape[1]]
    M1n, M2n, M3n = Wn[2], Wn[3], Wn[4]
    mb = jnp.zeros((3, _P), f32)
    mb = mb.at[0, :M1b.shape[0]].set(M1b)
    mb = mb.at[1, :M2b.shape[0]].set(M2b)
    mb = mb.at[2, :M3b.shape[0]].set(M3b)

    blk = 2000
    src = edge_index[0]
    dst = edge_index[1]
    cnt = _deg_counts(dst, N, E).reshape(_NC, -1, 1)
    h1p = _layer1_premul(x, W1n, cnt, blk)
    p1 = _aggregate(h1p, src, dst, N, E, W1n.shape[0])
    h2p = _layer2_premul(p1, h1p, cnt, b1[None, :], W2n, blk)
    p2 = _aggregate(h2p, src, dst, N, E, W2n.shape[0])
    return _final(p2, h2p, cnt, b2[None, :], M1n, M2n, M3n, mb, blk)
